# baseline (speedup 1.0000x reference)
import functools

import jax
import jax.numpy as jnp
from jax import lax
from jax.experimental import pallas as pl
from jax.experimental.pallas import tpu as pltpu
from jax.experimental.pallas import tpu_sc as plsc

B = 16384
D = 64
L = 16
NW = 32
RPW = B // NW
NCH = RPW // 128
MARGIN = 1.0

_mesh = plsc.VectorSubcoreMesh(core_axis_name="c", subcore_axis_name="s")


@functools.partial(
    pl.kernel,
    out_type=[
        jax.ShapeDtypeStruct((B, L), jnp.float32),
        jax.ShapeDtypeStruct((B, L), jnp.float32),
    ],
    mesh=_mesh,
    compiler_params=pltpu.CompilerParams(use_tc_tiling_on_sc=False),
    scratch_types=[
        pltpu.VMEM((6, NCH, 128), jnp.int32),
        pltpu.VMEM((RPW, D), jnp.float32),
        pltpu.VMEM((RPW, D), jnp.float32),
        pltpu.VMEM((RPW, D), jnp.float32),
        pltpu.VMEM((RPW, L), jnp.float32),
        pltpu.SemaphoreType.DMA,
    ],
)
def _sc_scores(idx_hbm, ent_hbm, rel_hbm, pos_out, neg_out,
               idx_v, h_v, r_v, t_v, o_v, sem):
    wid = lax.axis_index("s") * 2 + lax.axis_index("c")
    base = wid * RPW
    pltpu.sync_copy(idx_hbm.at[wid], idx_v)
    for side, out_hbm in ((0, pos_out), (1, neg_out)):
        copies = []
        for j in range(NCH):
            dst = pl.ds(j * 128, 128)
            copies.append(pltpu.async_copy(
                ent_hbm.at[idx_v.at[3 * side + 0, j]], h_v.at[dst], sem))
            copies.append(pltpu.async_copy(
                rel_hbm.at[idx_v.at[3 * side + 1, j]], r_v.at[dst], sem))
            copies.append(pltpu.async_copy(
                ent_hbm.at[idx_v.at[3 * side + 2, j]], t_v.at[dst], sem))
        for cp in copies:
            cp.wait()

        def row_body(i, carry):
            s = None
            for k in range(D // L):
                dsl = pl.ds(k * L, L)
                dv = h_v[i, dsl] + r_v[i, dsl] - t_v[i, dsl]
                sq = dv * dv
                s = sq if s is None else s + sq
            o_v[i, :] = s
            return carry

        lax.fori_loop(0, RPW, row_body, 0)
        pltpu.sync_copy(o_v, out_hbm.at[pl.ds(base, RPW)])


def _tc_loss(p_ref, n_ref, o_ref):
    sp = jnp.sqrt(jnp.sum(p_ref[...], axis=1))
    sn = jnp.sqrt(jnp.sum(n_ref[...], axis=1))
    hinge = jnp.maximum(MARGIN + sp - sn, 0.0)
    o_ref[0] = jnp.sum(hinge) * (1.0 / B)


_loss_call = pl.pallas_call(
    _tc_loss,
    out_shape=jax.ShapeDtypeStruct((1,), jnp.float32),
    out_specs=pl.BlockSpec(memory_space=pltpu.SMEM),
)


def kernel(pos_triples, neg_triples, entity_emb, relation_emb):
    pt = pos_triples.astype(jnp.int32)
    nt = neg_triples.astype(jnp.int32)
    idx = jnp.stack(
        [pt[:, 0], pt[:, 1], pt[:, 2], nt[:, 0], nt[:, 1], nt[:, 2]], axis=0)
    idx = idx.reshape(6, NW, NCH, 128).transpose(1, 0, 2, 3)
    sq_pos, sq_neg = _sc_scores(idx, entity_emb, relation_emb)
    loss = _loss_call(sq_pos, sq_neg)
    return loss[0]


# speedup vs baseline: 1.6084x; 1.6084x over previous
import functools

import jax
import jax.numpy as jnp
from jax import lax
from jax.experimental import pallas as pl
from jax.experimental.pallas import tpu as pltpu
from jax.experimental.pallas import tpu_sc as plsc

B = 16384
D = 64
L = 16
NW = 32
RPW = B // NW
CH = 128
NCH = RPW // CH
MARGIN = 1.0

_mesh = plsc.VectorSubcoreMesh(core_axis_name="c", subcore_axis_name="s")


@functools.partial(
    pl.kernel,
    out_type=[
        jax.ShapeDtypeStruct((B, L), jnp.float32),
        jax.ShapeDtypeStruct((B, L), jnp.float32),
    ],
    mesh=_mesh,
    scratch_types=[
        pltpu.VMEM((6 * NCH, CH), jnp.int32),
        pltpu.VMEM((CH, D), jnp.float32),
        pltpu.VMEM((CH, D), jnp.float32),
        pltpu.VMEM((CH, D), jnp.float32),
        pltpu.VMEM((CH, L), jnp.float32),
        pltpu.SemaphoreType.DMA,
    ],
)
def _sc_scores(idx_hbm, ent_hbm, rel_hbm, pos_out, neg_out,
               idx_v, h_v, r_v, t_v, part_v, sem):
    wid = lax.axis_index("s") * 2 + lax.axis_index("c")
    base = wid * RPW
    pltpu.sync_copy(idx_hbm.at[wid], idx_v)

    for side, out_hbm in ((0, pos_out), (1, neg_out)):
        for c in range(NCH):
            row_h = (3 * side + 0) * NCH + c
            row_r = (3 * side + 1) * NCH + c
            row_t = (3 * side + 2) * NCH + c

            def fire(g, carry):
                gsl = pl.ds(g * L, L)
                hv = idx_v[row_h, gsl]
                rv = idx_v[row_r, gsl]
                tv = idx_v[row_t, gsl]
                for j in range(L):
                    i = g * L + j
                    pltpu.async_copy(ent_hbm.at[hv[j]], h_v.at[i], sem)
                    pltpu.async_copy(rel_hbm.at[rv[j]], r_v.at[i], sem)
                    pltpu.async_copy(ent_hbm.at[tv[j]], t_v.at[i], sem)
                return carry

            lax.fori_loop(0, CH // L, fire, 0)
            pltpu.make_async_copy(ent_hbm.at[pl.ds(0, CH)], h_v, sem).wait()
            pltpu.make_async_copy(ent_hbm.at[pl.ds(0, CH)], r_v, sem).wait()
            pltpu.make_async_copy(ent_hbm.at[pl.ds(0, CH)], t_v, sem).wait()

            def dist(i, carry):
                s = None
                for k in range(D // L):
                    dsl = pl.ds(k * L, L)
                    dv = h_v[i, dsl] + r_v[i, dsl] - t_v[i, dsl]
                    sq = dv * dv
                    s = sq if s is None else s + sq
                part_v[i, :] = s
                return carry

            lax.fori_loop(0, CH, dist, 0)

            pltpu.sync_copy(part_v, out_hbm.at[pl.ds(base + c * CH, CH)])


def _tc_loss(p_ref, n_ref, o_ref):
    sp = jnp.sqrt(jnp.sum(p_ref[...], axis=1))
    sn = jnp.sqrt(jnp.sum(n_ref[...], axis=1))
    hinge = jnp.maximum(MARGIN + sp - sn, 0.0)
    o_ref[0] = jnp.sum(hinge) * (1.0 / B)


_loss_call = pl.pallas_call(
    _tc_loss,
    out_shape=jax.ShapeDtypeStruct((1,), jnp.float32),
    out_specs=pl.BlockSpec(memory_space=pltpu.SMEM),
)


def kernel(pos_triples, neg_triples, entity_emb, relation_emb):
    pt = pos_triples.astype(jnp.int32)
    nt = neg_triples.astype(jnp.int32)
    idx = jnp.stack(
        [pt[:, 0], pt[:, 1], pt[:, 2], nt[:, 0], nt[:, 1], nt[:, 2]], axis=0)
    idx = idx.reshape(6, NW, NCH, CH).transpose(1, 0, 2, 3)
    idx = idx.reshape(NW, 6 * NCH, CH)
    sq_pos, sq_neg = _sc_scores(idx, entity_emb, relation_emb)
    loss = _loss_call(sq_pos, sq_neg)
    return loss[0]
